# baseline (speedup 1.0000x reference)
import functools

import jax
import jax.numpy as jnp
from jax import lax
from jax.experimental import pallas as pl
from jax.experimental.pallas import tpu as pltpu
from jax.experimental.pallas import tpu_sc as plsc

N_NODES = 10000
NPAD = 10240
F_IN = 128
C = 128
TILES = 32
EPT = 10496
G = EPT // C
E_PAD = TILES * EPT
RB = 512
RPS = NPAD // 16

_SC_MESH = dict(core_axis_name="c", subcore_axis_name="s",
                num_cores=2, num_subcores=16)
_SC_PARAMS = pltpu.CompilerParams(
    needs_layout_passes=False, use_tc_tiling_on_sc=False)


def _edge_pass1(srcv, dstv, att, feat_eo, zden, znum):
    mesh = plsc.VectorSubcoreMesh(**_SC_MESH)

    @functools.partial(
        pl.kernel,
        out_type=[jax.ShapeDtypeStruct((2, NPAD, 8), jnp.float32),
                  jax.ShapeDtypeStruct((2, NPAD, 64), jnp.bfloat16)],
        mesh=mesh,
        compiler_params=_SC_PARAMS,
        scratch_types=[
            pltpu.VMEM((C,), jnp.int32),
            pltpu.VMEM((C,), jnp.int32),
            pltpu.VMEM((C, 16), jnp.float32),
            pltpu.VMEM((C, 16), jnp.float32),
            pltpu.VMEM((C, 64), jnp.float32),
            pltpu.VMEM((C, 8), jnp.float32),
            pltpu.VMEM((C, 64), jnp.bfloat16),
            pltpu.VMEM_SHARED((NPAD, 8), jnp.float32),
            pltpu.VMEM_SHARED((NPAD, 64), jnp.bfloat16),
            pltpu.SemaphoreType.DMA,
            pltpu.SemaphoreType.DMA,
        ],
    )
    def k(src_hbm, dst_hbm, a_hbm, h_hbm, zd_hbm, zn_hbm, oden_hbm, onum_hbm,
          v_is, v_id, v_gs, v_gd, v_h, v_db, v_nb, s_dt, s_nt, sem_a, sem_b):
        cid = lax.axis_index("c")
        sid = lax.axis_index("s")
        tile = cid * 16 + sid
        iot = lax.iota(jnp.int32, 16)

        for j in range(RPS // 128):
            r0 = sid * RPS + j * 128
            pltpu.sync_copy(zd_hbm, s_dt.at[pl.ds(r0, 128)])
            pltpu.sync_copy(zn_hbm, s_nt.at[pl.ds(r0, 128)])
        plsc.subcore_barrier()

        def chunk(g, _):
            ebase = tile * EPT + g * C
            cp1 = pltpu.async_copy(src_hbm.at[pl.ds(ebase, C)], v_is, sem_a)
            cp2 = pltpu.async_copy(dst_hbm.at[pl.ds(ebase, C)], v_id, sem_a)
            cp1.wait()
            cp2.wait()
            g1 = pltpu.async_copy(a_hbm.at[v_is], v_gs, sem_b)
            g2 = pltpu.async_copy(a_hbm.at[v_id], v_gd, sem_b)
            g3 = pltpu.async_copy(h_hbm.at[v_is], v_h, sem_b)
            g1.wait()
            g2.wait()
            g3.wait()

            def wp(t, _):
                pv = t * 16 + iot
                row = pv >> 3
                hd = pv & 7
                av = plsc.load_gather(v_gs, [row, hd])
                bv = plsc.load_gather(v_gd, [row, hd + 8])
                e = av + bv
                e = jnp.where(e > 0, e, 0.2 * e)
                plsc.store_scatter(v_db, [row, hd], jnp.exp(e))
                return 0
            lax.fori_loop(0, C * 8 // 16, wp, 0)

            def mp(c, _):
                cs = jnp.zeros((16,), jnp.int32) + c
                for k2 in range(2):
                    tmpl = (iot >> 2) + 4 * k2
                    wv = plsc.load_gather(v_db, [cs, tmpl])
                    av = v_h[c, pl.ds(16 * k2, 16)]
                    bv = v_h[c, pl.ds(32 + 16 * k2, 16)]
                    ia = plsc.bitcast(wv * av, jnp.int32)
                    ib = plsc.bitcast(wv * bv, jnp.int32)
                    ra = lax.shift_right_logical(ia + 0x8000, 16)
                    rb = (ib + 0x8000) & jnp.int32(-65536)
                    v_nb[c, pl.ds(32 * k2, 32)] = plsc.bitcast(
                        ra | rb, jnp.bfloat16)
                return 0
            lax.fori_loop(0, C, mp, 0)

            pltpu.sync_copy(v_db, s_dt.at[v_id], add=True)
            pltpu.sync_copy(v_nb, s_nt.at[v_id], add=True)
            return 0
        lax.fori_loop(0, G, chunk, 0)
        plsc.subcore_barrier()
        for j in range(RPS // 128):
            r0 = pl.multiple_of(sid * RPS + j * 128, 128)
            pltpu.sync_copy(s_dt.at[pl.ds(r0, 128)], v_db)
            pltpu.sync_copy(v_db, oden_hbm.at[cid, pl.ds(r0, 128)])
            pltpu.sync_copy(s_nt.at[pl.ds(r0, 128)], v_nb)
            pltpu.sync_copy(v_nb, onum_hbm.at[cid, pl.ds(r0, 128)])

    return k(srcv, dstv, att, feat_eo, zden, znum)


def _edge_pass2(srcv, dstv, att, feat):
    tblw = 32
    mesh = plsc.VectorSubcoreMesh(**_SC_MESH)

    @functools.partial(
        pl.kernel,
        out_type=jax.ShapeDtypeStruct((2, NPAD, tblw), jnp.float32),
        mesh=mesh,
        compiler_params=_SC_PARAMS,
        scratch_types=[
            pltpu.VMEM((C,), jnp.int32),
            pltpu.VMEM((C,), jnp.int32),
            pltpu.VMEM((C, 16), jnp.float32),
            pltpu.VMEM((C, 16), jnp.float32),
            pltpu.VMEM((C, 16), jnp.float32),
            pltpu.VMEM((C, tblw), jnp.float32),
            pltpu.VMEM((128, tblw), jnp.float32),
            pltpu.VMEM_SHARED((NPAD, tblw), jnp.float32),
            pltpu.SemaphoreType.DMA,
            pltpu.SemaphoreType.DMA,
        ],
    )
    def k(src_hbm, dst_hbm, a_hbm, h_hbm, out_hbm,
          v_is, v_id, v_gs, v_gd, v_h, v_cb, v_z, s_tbl, sem_a, sem_b):
        cid = lax.axis_index("c")
        sid = lax.axis_index("s")
        tile = cid * 16 + sid
        z16 = jnp.zeros((16,), jnp.float32)
        iot = lax.iota(jnp.int32, 16)

        def zero_rows(ref, rows):
            def zb(r, _):
                for o in range(tblw // 16):
                    ref[r, pl.ds(o * 16, 16)] = z16
                return 0
            lax.fori_loop(0, rows, zb, 0)

        zero_rows(v_z, 128)
        zero_rows(v_cb, C)
        for j in range(RPS // 128):
            pltpu.sync_copy(v_z, s_tbl.at[pl.ds(sid * RPS + j * 128, 128)])
        plsc.subcore_barrier()

        def chunk(g, _):
            ebase = tile * EPT + g * C
            cp1 = pltpu.async_copy(src_hbm.at[pl.ds(ebase, C)], v_is, sem_a)
            cp2 = pltpu.async_copy(dst_hbm.at[pl.ds(ebase, C)], v_id, sem_a)
            cp1.wait()
            cp2.wait()
            g1 = pltpu.async_copy(a_hbm.at[v_is], v_gs, sem_b)
            g2 = pltpu.async_copy(a_hbm.at[v_id], v_gd, sem_b)
            g3 = pltpu.async_copy(h_hbm.at[v_is], v_h, sem_b)
            g1.wait()
            g2.wait()
            g3.wait()

            def wp(t, _):
                row = t * 16 + iot
                hd = iot & 0
                av = plsc.load_gather(v_gs, [row, hd])
                bv = plsc.load_gather(v_gd, [row, hd + 8])
                e = av + bv
                e = jnp.where(e > 0, e, 0.2 * e)
                plsc.store_scatter(v_cb, [row, hd], jnp.exp(e))
                return 0
            lax.fori_loop(0, C // 16, wp, 0)

            def mp(c, _):
                cs = jnp.zeros((16,), jnp.int32) + c
                tmpl = iot >> 4
                wv = plsc.load_gather(v_cb, [cs, tmpl])
                hv = v_h[c, pl.ds(0, 16)]
                v_cb[c, pl.ds(8, 16)] = wv * hv
                return 0
            lax.fori_loop(0, C, mp, 0)

            pltpu.sync_copy(v_cb, s_tbl.at[v_id], add=True)
            return 0
        lax.fori_loop(0, G, chunk, 0)
        plsc.subcore_barrier()
        for j in range(RPS // 128):
            r0 = pl.multiple_of(sid * RPS + j * 128, 128)
            pltpu.sync_copy(s_tbl.at[pl.ds(r0, 128)], v_z)
            pltpu.sync_copy(v_z, out_hbm.at[cid, pl.ds(r0, 128)])

    return k(srcv, dstv, att, feat)


def _k1_body(x_ref, w_ref, s_ref, h_ref, a_ref):
    h = jnp.dot(x_ref[...], w_ref[...], preferred_element_type=jnp.float32)
    h_ref[...] = h
    a_ref[...] = jnp.dot(h, s_ref[...], preferred_element_type=jnp.float32)


def _k2_body(d_ref, n_ref, er_ref, b_ref, w2_ref, s2_ref, h2_ref, a2_ref):
    num = n_ref[0].astype(jnp.float32) + n_ref[1].astype(jnp.float32)
    den = jnp.dot(d_ref[0] + d_ref[1], er_ref[...],
                  preferred_element_type=jnp.float32)
    out1 = num / (den + 1e-16) + b_ref[...]
    out1 = jnp.where(out1 > 0, out1, 0.2 * out1)
    h2 = jnp.dot(out1, w2_ref[...], preferred_element_type=jnp.float32)
    h2_ref[...] = h2
    a2_ref[...] = jnp.dot(h2, s2_ref[...], preferred_element_type=jnp.float32)


def _k3_body(p_ref, b_ref, o_ref):
    t = p_ref[0] + p_ref[1]
    z = t[:, 8:24] / (t[:, 0:1] + 1e-16) + b_ref[...]
    m = jnp.max(z, axis=1, keepdims=True)
    o_ref[...] = z - m - jnp.log(jnp.sum(jnp.exp(z - m), axis=1, keepdims=True))


def kernel(x, edge_index, W1, as1, ad1, b1, W2, as2, ad2, b2):
    n = x.shape[0]
    loops = jnp.arange(n, dtype=jnp.int32)
    pad = jnp.full((E_PAD - edge_index.shape[1] - n,), N_NODES, jnp.int32)
    src = jnp.concatenate([edge_index[0].astype(jnp.int32), loops, pad])
    dst = jnp.concatenate([edge_index[1].astype(jnp.int32), loops, pad])
    x_pad = jnp.zeros((NPAD, F_IN), jnp.float32).at[:n].set(x)

    perm = [2 * i for i in range(32)] + [2 * i + 1 for i in range(32)]
    W1eo = W1[:, jnp.array(perm)]
    s1 = jnp.zeros((64, 16), jnp.float32)
    for hd in range(8):
        s1 = s1.at[hd * 8:(hd + 1) * 8, hd].set(as1[hd])
        s1 = s1.at[hd * 8:(hd + 1) * 8, 8 + hd].set(ad1[hd])
    s1eo = s1[jnp.array(perm), :]
    s2 = jnp.zeros((16, 16), jnp.float32).at[:, 0].set(as2[0]).at[:, 8].set(ad2[0])
    erep = jnp.repeat(jnp.eye(8, dtype=jnp.float32), 8, axis=1)
    zden = jnp.zeros((128, 8), jnp.float32)
    znum = jnp.zeros((128, 64), jnp.bfloat16)

    grid = (NPAD // RB,)
    h1, a1 = pl.pallas_call(
        _k1_body,
        grid=grid,
        in_specs=[
            pl.BlockSpec((RB, F_IN), lambda i: (i, 0)),
            pl.BlockSpec((F_IN, 64), lambda i: (0, 0)),
            pl.BlockSpec((64, 16), lambda i: (0, 0)),
        ],
        out_specs=[
            pl.BlockSpec((RB, 64), lambda i: (i, 0)),
            pl.BlockSpec((RB, 16), lambda i: (i, 0)),
        ],
        out_shape=[
            jax.ShapeDtypeStruct((NPAD, 64), jnp.float32),
            jax.ShapeDtypeStruct((NPAD, 16), jnp.float32),
        ],
    )(x_pad, W1eo, s1eo)

    den1, num1 = _edge_pass1(src, dst, a1, h1, zden, znum)

    h2, a2 = pl.pallas_call(
        _k2_body,
        grid=grid,
        in_specs=[
            pl.BlockSpec((2, RB, 8), lambda i: (0, i, 0)),
            pl.BlockSpec((2, RB, 64), lambda i: (0, i, 0)),
            pl.BlockSpec((8, 64), lambda i: (0, 0)),
            pl.BlockSpec((1, 64), lambda i: (0, 0)),
            pl.BlockSpec((64, 16), lambda i: (0, 0)),
            pl.BlockSpec((16, 16), lambda i: (0, 0)),
        ],
        out_specs=[
            pl.BlockSpec((RB, 16), lambda i: (i, 0)),
            pl.BlockSpec((RB, 16), lambda i: (i, 0)),
        ],
        out_shape=[
            jax.ShapeDtypeStruct((NPAD, 16), jnp.float32),
            jax.ShapeDtypeStruct((NPAD, 16), jnp.float32),
        ],
    )(den1, num1, erep, b1.reshape(1, 64), W2, s2)

    p2 = _edge_pass2(src, dst, a2, h2)

    out = pl.pallas_call(
        _k3_body,
        grid=grid,
        in_specs=[
            pl.BlockSpec((2, RB, 32), lambda i: (0, i, 0)),
            pl.BlockSpec((1, 16), lambda i: (0, 0)),
        ],
        out_specs=pl.BlockSpec((RB, 16), lambda i: (i, 0)),
        out_shape=jax.ShapeDtypeStruct((NPAD, 16), jnp.float32),
    )(p2, b2.reshape(1, 16))
    return out[:n]


# speedup vs baseline: 1.4076x; 1.4076x over previous
import functools

import jax
import jax.numpy as jnp
from jax import lax
from jax.experimental import pallas as pl
from jax.experimental.pallas import tpu as pltpu
from jax.experimental.pallas import tpu_sc as plsc

N_NODES = 10000
NPAD = 10240
F_IN = 128
C = 128
TILES = 32
EPT = 10496
G = EPT // C
E_PAD = TILES * EPT
RB = 512
RPS = NPAD // 16

_SC_MESH = dict(core_axis_name="c", subcore_axis_name="s",
                num_cores=2, num_subcores=16)
_SC_PARAMS = pltpu.CompilerParams(
    needs_layout_passes=False, use_tc_tiling_on_sc=False)


def _edge_pass(srcv, dstv, att, feat, heads, cols, tblw):
    kv = cols // 16
    span_sh = 3 if cols // heads == 8 else 4
    mesh = plsc.VectorSubcoreMesh(**_SC_MESH)

    @functools.partial(
        pl.kernel,
        out_type=jax.ShapeDtypeStruct((2, NPAD, tblw), jnp.float32),
        mesh=mesh,
        compiler_params=_SC_PARAMS,
        scratch_types=[
            [pltpu.VMEM((C,), jnp.int32)] * 2,
            [pltpu.VMEM((C,), jnp.int32)] * 2,
            [pltpu.VMEM((C, 16), jnp.float32)] * 2,
            [pltpu.VMEM((C, 16), jnp.float32)] * 2,
            [pltpu.VMEM((C, cols), jnp.float32)] * 2,
            [pltpu.VMEM((C, tblw), jnp.float32)] * 2,
            pltpu.VMEM((128, tblw), jnp.float32),
            pltpu.VMEM_SHARED((NPAD, tblw), jnp.float32),
            [pltpu.SemaphoreType.DMA] * 2,
            [pltpu.SemaphoreType.DMA] * 2,
            [pltpu.SemaphoreType.DMA] * 2,
        ],
    )
    def k(src_hbm, dst_hbm, a_hbm, h_hbm, out_hbm,
          v_is, v_id, v_gs, v_gd, v_h, v_cb, v_z, s_tbl,
          sem_i, sem_g, sem_s):
        cid = lax.axis_index("c")
        sid = lax.axis_index("s")
        tile = cid * 16 + sid
        z16 = jnp.zeros((16,), jnp.float32)
        zi16 = jnp.zeros((16,), jnp.int32)
        iot = lax.iota(jnp.int32, 16)
        tmpls = [(k2 * 16 + iot) >> span_sh for k2 in range(kv)]

        def ebase(g):
            return tile * EPT + g * C

        def istart(g, p):
            pltpu.async_copy(src_hbm.at[pl.ds(ebase(g), C)], v_is[p], sem_i[p])
            pltpu.async_copy(dst_hbm.at[pl.ds(ebase(g), C)], v_id[p], sem_i[p])

        def iwait(g, p):
            pltpu.make_async_copy(
                src_hbm.at[pl.ds(ebase(g), C)], v_is[p], sem_i[p]).wait()
            pltpu.make_async_copy(
                dst_hbm.at[pl.ds(ebase(g), C)], v_id[p], sem_i[p]).wait()

        def gstart(p):
            pltpu.async_copy(a_hbm.at[v_is[p]], v_gs[p], sem_g[p])
            pltpu.async_copy(a_hbm.at[v_id[p]], v_gd[p], sem_g[p])
            pltpu.async_copy(h_hbm.at[v_is[p]], v_h[p], sem_g[p])

        def gwait(p):
            pltpu.make_async_copy(a_hbm.at[v_is[p]], v_gs[p], sem_g[p]).wait()
            pltpu.make_async_copy(a_hbm.at[v_id[p]], v_gd[p], sem_g[p]).wait()
            pltpu.make_async_copy(h_hbm.at[v_is[p]], v_h[p], sem_g[p]).wait()

        def sstart(p):
            pltpu.async_copy(v_cb[p], s_tbl.at[v_id[p]], sem_s[p], add=True)

        def swait(p):
            pltpu.make_async_copy(v_cb[p], s_tbl.at[v_id[p]], sem_s[p]).wait()

        def zero_rows(ref, rows):
            def zb(r, _):
                for o in range(tblw // 16):
                    ref[r, pl.ds(o * 16, 16)] = z16
                return 0
            lax.fori_loop(0, rows, zb, 0)

        zero_rows(v_z, 128)
        for p in range(2):
            zero_rows(v_cb[p], C)
            def zid(r, _):
                v_id[p][pl.ds(r * 16, 16)] = zi16
                return 0
            lax.fori_loop(0, C // 16, zid, 0)
        for j in range(RPS // 128):
            pltpu.sync_copy(v_z, s_tbl.at[pl.ds(sid * RPS + j * 128, 128)])
        plsc.subcore_barrier()
        sstart(1)
        istart(0, 0)
        iwait(0, 0)
        gstart(0)

        def compute(p):
            v_gs_p, v_gd_p, v_h_p, v_cb_p = v_gs[p], v_gd[p], v_h[p], v_cb[p]

            def wp(t, _):
                pv = t * 16 + iot
                if heads == 8:
                    row = pv >> 3
                    hd = pv & 7
                else:
                    row = pv
                    hd = iot & 0
                av = plsc.load_gather(v_gs_p, [row, hd])
                bv = plsc.load_gather(v_gd_p, [row, hd + 8])
                e = av + bv
                e = jnp.where(e > 0, e, 0.2 * e)
                plsc.store_scatter(v_cb_p, [row, hd], jnp.exp(e))
                return 0
            lax.fori_loop(0, C * heads // 16, wp, 0, unroll=2)

            def mp(c, _):
                cs = jnp.zeros((16,), jnp.int32) + c
                for k2 in range(kv):
                    wv = plsc.load_gather(v_cb_p, [cs, tmpls[k2]])
                    hv = v_h_p[c, pl.ds(k2 * 16, 16)]
                    v_cb_p[c, pl.ds(8 + k2 * 16, 16)] = wv * hv
                return 0
            lax.fori_loop(0, C, mp, 0, unroll=2)

        def step(g, p, q):
            swait(q)
            gnxt = jnp.minimum(g + 1, G - 1)
            istart(gnxt, q)
            gwait(p)
            iwait(gnxt, q)
            gstart(q)
            compute(p)
            sstart(p)

        def pair(i, _):
            step(2 * i, 0, 1)
            step(2 * i + 1, 1, 0)
            return 0
        lax.fori_loop(0, G // 2, pair, 0)
        swait(1)
        gwait(0)
        plsc.subcore_barrier()
        for j in range(RPS // 128):
            r0 = pl.multiple_of(sid * RPS + j * 128, 128)
            pltpu.sync_copy(s_tbl.at[pl.ds(r0, 128)], v_z)
            pltpu.sync_copy(v_z, out_hbm.at[cid, pl.ds(r0, 128)])

    return k(srcv, dstv, att, feat)


def _k1_body(x_ref, w_ref, s_ref, h_ref, a_ref):
    h = jnp.dot(x_ref[...], w_ref[...], preferred_element_type=jnp.float32)
    h_ref[...] = h
    a_ref[...] = jnp.dot(h, s_ref[...], preferred_element_type=jnp.float32)


def _k2_body(p_ref, er_ref, b_ref, w2_ref, s2_ref, h2_ref, a2_ref):
    t = p_ref[0] + p_ref[1]
    den = jnp.dot(t[:, 0:8], er_ref[...], preferred_element_type=jnp.float32)
    out1 = t[:, 8:72] / (den + 1e-16) + b_ref[...]
    out1 = jnp.where(out1 > 0, out1, 0.2 * out1)
    h2 = jnp.dot(out1, w2_ref[...], preferred_element_type=jnp.float32)
    h2_ref[...] = h2
    a2_ref[...] = jnp.dot(h2, s2_ref[...], preferred_element_type=jnp.float32)


def _k3_body(p_ref, b_ref, o_ref):
    t = p_ref[0] + p_ref[1]
    z = t[:, 8:24] / (t[:, 0:1] + 1e-16) + b_ref[...]
    m = jnp.max(z, axis=1, keepdims=True)
    o_ref[...] = z - m - jnp.log(jnp.sum(jnp.exp(z - m), axis=1, keepdims=True))


def kernel(x, edge_index, W1, as1, ad1, b1, W2, as2, ad2, b2):
    n = x.shape[0]
    loops = jnp.arange(n, dtype=jnp.int32)
    pad = jnp.full((E_PAD - edge_index.shape[1] - n,), N_NODES, jnp.int32)
    src = jnp.concatenate([edge_index[0].astype(jnp.int32), loops, pad])
    dst = jnp.concatenate([edge_index[1].astype(jnp.int32), loops, pad])
    x_pad = jnp.zeros((NPAD, F_IN), jnp.float32).at[:n].set(x)

    s1 = jnp.zeros((64, 16), jnp.float32)
    for hd in range(8):
        s1 = s1.at[hd * 8:(hd + 1) * 8, hd].set(as1[hd])
        s1 = s1.at[hd * 8:(hd + 1) * 8, 8 + hd].set(ad1[hd])
    s2 = jnp.zeros((16, 16), jnp.float32).at[:, 0].set(as2[0]).at[:, 8].set(ad2[0])
    erep = jnp.repeat(jnp.eye(8, dtype=jnp.float32), 8, axis=1)

    grid = (NPAD // RB,)
    h1, a1 = pl.pallas_call(
        _k1_body,
        grid=grid,
        in_specs=[
            pl.BlockSpec((RB, F_IN), lambda i: (i, 0)),
            pl.BlockSpec((F_IN, 64), lambda i: (0, 0)),
            pl.BlockSpec((64, 16), lambda i: (0, 0)),
        ],
        out_specs=[
            pl.BlockSpec((RB, 64), lambda i: (i, 0)),
            pl.BlockSpec((RB, 16), lambda i: (i, 0)),
        ],
        out_shape=[
            jax.ShapeDtypeStruct((NPAD, 64), jnp.float32),
            jax.ShapeDtypeStruct((NPAD, 16), jnp.float32),
        ],
    )(x_pad, W1, s1)

    p1 = _edge_pass(src, dst, a1, h1, heads=8, cols=64, tblw=80)

    h2, a2 = pl.pallas_call(
        _k2_body,
        grid=grid,
        in_specs=[
            pl.BlockSpec((2, RB, 80), lambda i: (0, i, 0)),
            pl.BlockSpec((8, 64), lambda i: (0, 0)),
            pl.BlockSpec((1, 64), lambda i: (0, 0)),
            pl.BlockSpec((64, 16), lambda i: (0, 0)),
            pl.BlockSpec((16, 16), lambda i: (0, 0)),
        ],
        out_specs=[
            pl.BlockSpec((RB, 16), lambda i: (i, 0)),
            pl.BlockSpec((RB, 16), lambda i: (i, 0)),
        ],
        out_shape=[
            jax.ShapeDtypeStruct((NPAD, 16), jnp.float32),
            jax.ShapeDtypeStruct((NPAD, 16), jnp.float32),
        ],
    )(p1, erep, b1.reshape(1, 64), W2, s2)

    p2 = _edge_pass(src, dst, a2, h2, heads=1, cols=16, tblw=32)

    out = pl.pallas_call(
        _k3_body,
        grid=grid,
        in_specs=[
            pl.BlockSpec((2, RB, 32), lambda i: (0, i, 0)),
            pl.BlockSpec((1, 16), lambda i: (0, 0)),
        ],
        out_specs=pl.BlockSpec((RB, 16), lambda i: (i, 0)),
        out_shape=jax.ShapeDtypeStruct((NPAD, 16), jnp.float32),
    )(p2, b2.reshape(1, 16))
    return out[:n]


# speedup vs baseline: 1.4651x; 1.0408x over previous
import functools

import jax
import jax.numpy as jnp
from jax import lax
from jax.experimental import pallas as pl
from jax.experimental.pallas import tpu as pltpu
from jax.experimental.pallas import tpu_sc as plsc

N_NODES = 10000
NPAD = 10240
F_IN = 128
C = 128
TILES = 32
EPT = 10496
G = EPT // C
E_PAD = TILES * EPT
RB = 512
RPS = NPAD // 16

_SC_MESH = dict(core_axis_name="c", subcore_axis_name="s",
                num_cores=2, num_subcores=16)
_SC_PARAMS = pltpu.CompilerParams(
    needs_layout_passes=False, use_tc_tiling_on_sc=False)


def _edge_pass(srcv, dstv, att, feat, heads, cols, tblw):
    kv = cols // 16
    span_sh = 3 if cols // heads == 8 else 4
    mesh = plsc.VectorSubcoreMesh(**_SC_MESH)

    @functools.partial(
        pl.kernel,
        out_type=jax.ShapeDtypeStruct((2, NPAD, tblw), jnp.float32),
        mesh=mesh,
        compiler_params=_SC_PARAMS,
        scratch_types=[
            [pltpu.VMEM((C,), jnp.int32)] * 2,
            [pltpu.VMEM((C,), jnp.int32)] * 2,
            [pltpu.VMEM((C, 16), jnp.float32)] * 2,
            [pltpu.VMEM((C, 16), jnp.float32)] * 2,
            [pltpu.VMEM((C, cols), jnp.float32)] * 2,
            [pltpu.VMEM((C, tblw), jnp.float32)] * 2,
            pltpu.VMEM((128, tblw), jnp.float32),
            pltpu.VMEM_SHARED((NPAD, tblw), jnp.float32),
            [pltpu.SemaphoreType.DMA] * 2,
            [pltpu.SemaphoreType.DMA] * 2,
            [pltpu.SemaphoreType.DMA] * 2,
        ],
    )
    def k(src_hbm, dst_hbm, a_hbm, h_hbm, out_hbm,
          v_is, v_id, v_gs, v_gd, v_h, v_cb, v_z, s_tbl,
          sem_i, sem_g, sem_s):
        cid = lax.axis_index("c")
        sid = lax.axis_index("s")
        tile = cid * 16 + sid
        z16 = jnp.zeros((16,), jnp.float32)
        zi16 = jnp.zeros((16,), jnp.int32)
        iot = lax.iota(jnp.int32, 16)
        tmpls = [(k2 * 16 + iot) >> span_sh for k2 in range(kv)]

        def ebase(g):
            return tile * EPT + g * C

        def istart(g, p):
            pltpu.async_copy(src_hbm.at[pl.ds(ebase(g), C)], v_is[p], sem_i[p])
            pltpu.async_copy(dst_hbm.at[pl.ds(ebase(g), C)], v_id[p], sem_i[p])

        def iwait(g, p):
            pltpu.make_async_copy(
                src_hbm.at[pl.ds(ebase(g), C)], v_is[p], sem_i[p]).wait()
            pltpu.make_async_copy(
                dst_hbm.at[pl.ds(ebase(g), C)], v_id[p], sem_i[p]).wait()

        def gstart(p):
            pltpu.async_copy(a_hbm.at[v_is[p]], v_gs[p], sem_g[p])
            pltpu.async_copy(a_hbm.at[v_id[p]], v_gd[p], sem_g[p])
            pltpu.async_copy(h_hbm.at[v_is[p]], v_h[p], sem_g[p])

        def gwait(p):
            pltpu.make_async_copy(a_hbm.at[v_is[p]], v_gs[p], sem_g[p]).wait()
            pltpu.make_async_copy(a_hbm.at[v_id[p]], v_gd[p], sem_g[p]).wait()
            pltpu.make_async_copy(h_hbm.at[v_is[p]], v_h[p], sem_g[p]).wait()

        def sstart(p):
            pltpu.async_copy(v_cb[p], s_tbl.at[v_id[p]], sem_s[p], add=True)

        def swait(p):
            pltpu.make_async_copy(v_cb[p], s_tbl.at[v_id[p]], sem_s[p]).wait()

        def zero_rows(ref, rows):
            def zb(r, _):
                for o in range(tblw // 16):
                    ref[r, pl.ds(o * 16, 16)] = z16
                return 0
            lax.fori_loop(0, rows, zb, 0)

        zero_rows(v_z, 128)
        for p in range(2):
            zero_rows(v_cb[p], C)
            def zid(r, _):
                v_id[p][pl.ds(r * 16, 16)] = zi16
                return 0
            lax.fori_loop(0, C // 16, zid, 0)
        for j in range(RPS // 128):
            pltpu.sync_copy(v_z, s_tbl.at[pl.ds(sid * RPS + j * 128, 128)])
        plsc.subcore_barrier()
        sstart(1)
        istart(0, 0)
        iwait(0, 0)
        gstart(0)

        def compute(p):
            v_gs_p, v_gd_p, v_h_p, v_cb_p = v_gs[p], v_gd[p], v_h[p], v_cb[p]

            def wp(t, _):
                pv = t * 16 + iot
                if heads == 8:
                    row = pv >> 3
                    hd = pv & 7
                else:
                    row = pv
                    hd = iot & 0
                av = plsc.load_gather(v_gs_p, [row, hd])
                bv = plsc.load_gather(v_gd_p, [row, hd + 8])
                e = av + bv
                e = jnp.where(e > 0, e, 0.2 * e)
                plsc.store_scatter(v_cb_p, [row, hd], jnp.exp(e))
                return 0
            lax.fori_loop(0, C * heads // 16, wp, 0, unroll=2)

            def mp(c, _):
                cs = jnp.zeros((16,), jnp.int32) + c
                for k2 in range(kv):
                    wv = plsc.load_gather(v_cb_p, [cs, tmpls[k2]])
                    hv = v_h_p[c, pl.ds(k2 * 16, 16)]
                    v_cb_p[c, pl.ds(8 + k2 * 16, 16)] = wv * hv
                return 0
            lax.fori_loop(0, C, mp, 0, unroll=2)

        def step(g, p, q):
            swait(q)
            gnxt = jnp.minimum(g + 1, G - 1)
            istart(gnxt, q)
            gwait(p)
            iwait(gnxt, q)
            gstart(q)
            compute(p)
            sstart(p)

        def pair(i, _):
            step(2 * i, 0, 1)
            step(2 * i + 1, 1, 0)
            return 0
        lax.fori_loop(0, G // 2, pair, 0)
        swait(1)
        gwait(0)
        plsc.subcore_barrier()
        for j in range(RPS // 128):
            r0 = pl.multiple_of(sid * RPS + j * 128, 128)
            pltpu.sync_copy(s_tbl.at[pl.ds(r0, 128)], v_z)
            pltpu.sync_copy(v_z, out_hbm.at[cid, pl.ds(r0, 128)])

    return k(srcv, dstv, att, feat)


def _edge_pass1bf(srcv, dstv, att, feat_eo, zden, znum):
    mesh = plsc.VectorSubcoreMesh(**_SC_MESH)

    @functools.partial(
        pl.kernel,
        out_type=[jax.ShapeDtypeStruct((2, NPAD, 8), jnp.float32),
                  jax.ShapeDtypeStruct((2, NPAD, 64), jnp.bfloat16)],
        mesh=mesh,
        compiler_params=_SC_PARAMS,
        scratch_types=[
            [pltpu.VMEM((C,), jnp.int32)] * 2,
            [pltpu.VMEM((C,), jnp.int32)] * 2,
            [pltpu.VMEM((C, 16), jnp.float32)] * 2,
            [pltpu.VMEM((C, 16), jnp.float32)] * 2,
            [pltpu.VMEM((C, 64), jnp.float32)] * 2,
            [pltpu.VMEM((C, 8), jnp.float32)] * 2,
            [pltpu.VMEM((C, 64), jnp.bfloat16)] * 2,
            [pltpu.SemaphoreType.DMA] * 2,
            [pltpu.SemaphoreType.DMA] * 2,
            [pltpu.SemaphoreType.DMA] * 2,
            pltpu.VMEM_SHARED((NPAD, 8), jnp.float32),
            pltpu.VMEM_SHARED((NPAD, 64), jnp.bfloat16),
        ],
    )
    def k(src_hbm, dst_hbm, a_hbm, h_hbm, zd_hbm, zn_hbm, oden_hbm, onum_hbm,
          v_is, v_id, v_gs, v_gd, v_h, v_db, v_nb,
          sem_i, sem_g, sem_s, s_dt, s_nt):
        cid = lax.axis_index("c")
        sid = lax.axis_index("s")
        tile = cid * 16 + sid
        z16 = jnp.zeros((16,), jnp.float32)
        zi16 = jnp.zeros((16,), jnp.int32)
        zb32 = jnp.zeros((32,), jnp.bfloat16)
        iot = lax.iota(jnp.int32, 16)

        def ebase(g):
            return tile * EPT + g * C

        def istart(g, p):
            pltpu.async_copy(src_hbm.at[pl.ds(ebase(g), C)], v_is[p], sem_i[p])
            pltpu.async_copy(dst_hbm.at[pl.ds(ebase(g), C)], v_id[p], sem_i[p])

        def iwait(g, p):
            pltpu.make_async_copy(
                src_hbm.at[pl.ds(ebase(g), C)], v_is[p], sem_i[p]).wait()
            pltpu.make_async_copy(
                dst_hbm.at[pl.ds(ebase(g), C)], v_id[p], sem_i[p]).wait()

        def gstart(p):
            pltpu.async_copy(a_hbm.at[v_is[p]], v_gs[p], sem_g[p])
            pltpu.async_copy(a_hbm.at[v_id[p]], v_gd[p], sem_g[p])
            pltpu.async_copy(h_hbm.at[v_is[p]], v_h[p], sem_g[p])

        def gwait(p):
            pltpu.make_async_copy(a_hbm.at[v_is[p]], v_gs[p], sem_g[p]).wait()
            pltpu.make_async_copy(a_hbm.at[v_id[p]], v_gd[p], sem_g[p]).wait()
            pltpu.make_async_copy(h_hbm.at[v_is[p]], v_h[p], sem_g[p]).wait()

        def sstart(p):
            pltpu.async_copy(v_db[p], s_dt.at[v_id[p]], sem_s[p], add=True)
            pltpu.async_copy(v_nb[p], s_nt.at[v_id[p]], sem_s[p], add=True)

        def swait(p):
            pltpu.make_async_copy(v_db[p], s_dt.at[v_id[p]], sem_s[p]).wait()
            pltpu.make_async_copy(v_nb[p], s_nt.at[v_id[p]], sem_s[p]).wait()

        for j in range(RPS // 128):
            r0 = sid * RPS + j * 128
            pltpu.sync_copy(zd_hbm, s_dt.at[pl.ds(r0, 128)])
            pltpu.sync_copy(zn_hbm, s_nt.at[pl.ds(r0, 128)])
        def zb1(t, _):
            pv = t * 16 + iot
            plsc.store_scatter(v_db[1], [pv >> 3, pv & 7], z16)
            return 0
        lax.fori_loop(0, C * 8 // 16, zb1, 0)
        def zb2(r, _):
            v_nb[1][r, pl.ds(0, 32)] = zb32
            v_nb[1][r, pl.ds(32, 32)] = zb32
            return 0
        lax.fori_loop(0, C, zb2, 0)
        def zb3(r, _):
            v_id[1][pl.ds(r * 16, 16)] = zi16
            return 0
        lax.fori_loop(0, C // 16, zb3, 0)
        plsc.subcore_barrier()
        sstart(1)
        istart(0, 0)
        iwait(0, 0)
        gstart(0)

        def compute(p):
            v_gs_p, v_gd_p, v_h_p = v_gs[p], v_gd[p], v_h[p]
            v_db_p, v_nb_p = v_db[p], v_nb[p]

            def wp(t, _):
                pv = t * 16 + iot
                row = pv >> 3
                hd = pv & 7
                av = plsc.load_gather(v_gs_p, [row, hd])
                bv = plsc.load_gather(v_gd_p, [row, hd + 8])
                e = av + bv
                e = jnp.where(e > 0, e, 0.2 * e)
                plsc.store_scatter(v_db_p, [row, hd], jnp.exp(e))
                return 0
            lax.fori_loop(0, C * 8 // 16, wp, 0, unroll=2)

            def mp(c, _):
                cs = jnp.zeros((16,), jnp.int32) + c
                for k2 in range(2):
                    tmpl = (iot >> 2) + 4 * k2
                    wv = plsc.load_gather(v_db_p, [cs, tmpl])
                    av = v_h_p[c, pl.ds(16 * k2, 16)]
                    bv = v_h_p[c, pl.ds(32 + 16 * k2, 16)]
                    ia = plsc.bitcast(wv * av, jnp.int32)
                    ib = plsc.bitcast(wv * bv, jnp.int32)
                    ra = lax.shift_right_logical(ia + 0x8000, 16)
                    rb = (ib + 0x8000) & jnp.int32(-65536)
                    v_nb_p[c, pl.ds(32 * k2, 32)] = plsc.bitcast(
                        ra | rb, jnp.bfloat16)
                return 0
            lax.fori_loop(0, C, mp, 0, unroll=2)

        def step(g, p, q):
            swait(q)
            gnxt = jnp.minimum(g + 1, G - 1)
            istart(gnxt, q)
            gwait(p)
            iwait(gnxt, q)
            gstart(q)
            compute(p)
            sstart(p)

        def pair(i, _):
            step(2 * i, 0, 1)
            step(2 * i + 1, 1, 0)
            return 0
        lax.fori_loop(0, G // 2, pair, 0)
        swait(1)
        gwait(0)
        plsc.subcore_barrier()
        for j in range(RPS // 128):
            r0 = pl.multiple_of(sid * RPS + j * 128, 128)
            pltpu.sync_copy(s_dt.at[pl.ds(r0, 128)], v_db[0])
            pltpu.sync_copy(v_db[0], oden_hbm.at[cid, pl.ds(r0, 128)])
            pltpu.sync_copy(s_nt.at[pl.ds(r0, 128)], v_nb[0])
            pltpu.sync_copy(v_nb[0], onum_hbm.at[cid, pl.ds(r0, 128)])

    return k(srcv, dstv, att, feat_eo, zden, znum)


def _k1_body(x_ref, w_ref, s_ref, h_ref, a_ref):
    h = jnp.dot(x_ref[...], w_ref[...], preferred_element_type=jnp.float32)
    h_ref[...] = h
    a_ref[...] = jnp.dot(h, s_ref[...], preferred_element_type=jnp.float32)


def _k2_body(d_ref, n_ref, er_ref, b_ref, w2_ref, s2_ref, h2_ref, a2_ref):
    num = n_ref[0].astype(jnp.float32) + n_ref[1].astype(jnp.float32)
    den = jnp.dot(d_ref[0] + d_ref[1], er_ref[...],
                  preferred_element_type=jnp.float32)
    out1 = num / (den + 1e-16) + b_ref[...]
    out1 = jnp.where(out1 > 0, out1, 0.2 * out1)
    h2 = jnp.dot(out1, w2_ref[...], preferred_element_type=jnp.float32)
    h2_ref[...] = h2
    a2_ref[...] = jnp.dot(h2, s2_ref[...], preferred_element_type=jnp.float32)


def _k3_body(p_ref, b_ref, o_ref):
    t = p_ref[0] + p_ref[1]
    z = t[:, 8:24] / (t[:, 0:1] + 1e-16) + b_ref[...]
    m = jnp.max(z, axis=1, keepdims=True)
    o_ref[...] = z - m - jnp.log(jnp.sum(jnp.exp(z - m), axis=1, keepdims=True))


def kernel(x, edge_index, W1, as1, ad1, b1, W2, as2, ad2, b2):
    n = x.shape[0]
    loops = jnp.arange(n, dtype=jnp.int32)
    pad = jnp.full((E_PAD - edge_index.shape[1] - n,), N_NODES, jnp.int32)
    src = jnp.concatenate([edge_index[0].astype(jnp.int32), loops, pad])
    dst = jnp.concatenate([edge_index[1].astype(jnp.int32), loops, pad])
    x_pad = jnp.zeros((NPAD, F_IN), jnp.float32).at[:n].set(x)

    perm = jnp.array([2 * i for i in range(32)] + [2 * i + 1 for i in range(32)])
    W1eo = W1[:, perm]
    s1 = jnp.zeros((64, 16), jnp.float32)
    for hd in range(8):
        s1 = s1.at[hd * 8:(hd + 1) * 8, hd].set(as1[hd])
        s1 = s1.at[hd * 8:(hd + 1) * 8, 8 + hd].set(ad1[hd])
    s1eo = s1[perm, :]
    s2 = jnp.zeros((16, 16), jnp.float32).at[:, 0].set(as2[0]).at[:, 8].set(ad2[0])
    erep = jnp.repeat(jnp.eye(8, dtype=jnp.float32), 8, axis=1)
    zden = jnp.zeros((128, 8), jnp.float32)
    znum = jnp.zeros((128, 64), jnp.bfloat16)

    grid = (NPAD // RB,)
    h1, a1 = pl.pallas_call(
        _k1_body,
        grid=grid,
        in_specs=[
            pl.BlockSpec((RB, F_IN), lambda i: (i, 0)),
            pl.BlockSpec((F_IN, 64), lambda i: (0, 0)),
            pl.BlockSpec((64, 16), lambda i: (0, 0)),
        ],
        out_specs=[
            pl.BlockSpec((RB, 64), lambda i: (i, 0)),
            pl.BlockSpec((RB, 16), lambda i: (i, 0)),
        ],
        out_shape=[
            jax.ShapeDtypeStruct((NPAD, 64), jnp.float32),
            jax.ShapeDtypeStruct((NPAD, 16), jnp.float32),
        ],
    )(x_pad, W1eo, s1eo)

    den1, num1 = _edge_pass1bf(src, dst, a1, h1, zden, znum)

    h2, a2 = pl.pallas_call(
        _k2_body,
        grid=grid,
        in_specs=[
            pl.BlockSpec((2, RB, 8), lambda i: (0, i, 0)),
            pl.BlockSpec((2, RB, 64), lambda i: (0, i, 0)),
            pl.BlockSpec((8, 64), lambda i: (0, 0)),
            pl.BlockSpec((1, 64), lambda i: (0, 0)),
            pl.BlockSpec((64, 16), lambda i: (0, 0)),
            pl.BlockSpec((16, 16), lambda i: (0, 0)),
        ],
        out_specs=[
            pl.BlockSpec((RB, 16), lambda i: (i, 0)),
            pl.BlockSpec((RB, 16), lambda i: (i, 0)),
        ],
        out_shape=[
            jax.ShapeDtypeStruct((NPAD, 16), jnp.float32),
            jax.ShapeDtypeStruct((NPAD, 16), jnp.float32),
        ],
    )(den1, num1, erep, b1.reshape(1, 64), W2, s2)

    p2 = _edge_pass(src, dst, a2, h2, heads=1, cols=16, tblw=32)

    out = pl.pallas_call(
        _k3_body,
        grid=grid,
        in_specs=[
            pl.BlockSpec((2, RB, 32), lambda i: (0, i, 0)),
            pl.BlockSpec((1, 16), lambda i: (0, 0)),
        ],
        out_specs=pl.BlockSpec((RB, 16), lambda i: (i, 0)),
        out_shape=jax.ShapeDtypeStruct((NPAD, 16), jnp.float32),
    )(p2, b2.reshape(1, 16))
    return out[:n]


# speedup vs baseline: 1.4671x; 1.0013x over previous
import functools

import jax
import jax.numpy as jnp
from jax import lax
from jax.experimental import pallas as pl
from jax.experimental.pallas import tpu as pltpu
from jax.experimental.pallas import tpu_sc as plsc

N_NODES = 10000
NPAD = 10240
F_IN = 128
C = 128
TILES = 32
EPT = 10496
G = EPT // C
E_PAD = TILES * EPT
RB = 512
RPS = NPAD // 16

_SC_MESH = dict(core_axis_name="c", subcore_axis_name="s",
                num_cores=2, num_subcores=16)
_SC_PARAMS = pltpu.CompilerParams(
    needs_layout_passes=False, use_tc_tiling_on_sc=False)


def _edge_pass(srcv, dstv, att, feat, heads, cols, tblw):
    kv = cols // 16
    span_sh = 3 if cols // heads == 8 else 4
    mesh = plsc.VectorSubcoreMesh(**_SC_MESH)

    @functools.partial(
        pl.kernel,
        out_type=jax.ShapeDtypeStruct((2, NPAD, tblw), jnp.float32),
        mesh=mesh,
        compiler_params=_SC_PARAMS,
        scratch_types=[
            [pltpu.VMEM((C,), jnp.int32)] * 2,
            [pltpu.VMEM((C,), jnp.int32)] * 2,
            [pltpu.VMEM((C, 16), jnp.float32)] * 2,
            [pltpu.VMEM((C, 16), jnp.float32)] * 2,
            [pltpu.VMEM((C, cols), jnp.float32)] * 2,
            [pltpu.VMEM((C, tblw), jnp.float32)] * 2,
            pltpu.VMEM((128, tblw), jnp.float32),
            pltpu.VMEM_SHARED((NPAD, tblw), jnp.float32),
            [pltpu.SemaphoreType.DMA] * 2,
            [pltpu.SemaphoreType.DMA] * 2,
            [pltpu.SemaphoreType.DMA] * 2,
        ],
    )
    def k(src_hbm, dst_hbm, a_hbm, h_hbm, out_hbm,
          v_is, v_id, v_gs, v_gd, v_h, v_cb, v_z, s_tbl,
          sem_i, sem_g, sem_s):
        cid = lax.axis_index("c")
        sid = lax.axis_index("s")
        tile = cid * 16 + sid
        z16 = jnp.zeros((16,), jnp.float32)
        zi16 = jnp.zeros((16,), jnp.int32)
        iot = lax.iota(jnp.int32, 16)
        tmpls = [(k2 * 16 + iot) >> span_sh for k2 in range(kv)]

        def ebase(g):
            return tile * EPT + g * C

        def istart(g, p):
            pltpu.async_copy(src_hbm.at[pl.ds(ebase(g), C)], v_is[p], sem_i[p])
            pltpu.async_copy(dst_hbm.at[pl.ds(ebase(g), C)], v_id[p], sem_i[p])

        def iwait(g, p):
            pltpu.make_async_copy(
                src_hbm.at[pl.ds(ebase(g), C)], v_is[p], sem_i[p]).wait()
            pltpu.make_async_copy(
                dst_hbm.at[pl.ds(ebase(g), C)], v_id[p], sem_i[p]).wait()

        def gstart(p):
            pltpu.async_copy(a_hbm.at[v_is[p]], v_gs[p], sem_g[p])
            pltpu.async_copy(a_hbm.at[v_id[p]], v_gd[p], sem_g[p])
            pltpu.async_copy(h_hbm.at[v_is[p]], v_h[p], sem_g[p])

        def gwait(p):
            pltpu.make_async_copy(a_hbm.at[v_is[p]], v_gs[p], sem_g[p]).wait()
            pltpu.make_async_copy(a_hbm.at[v_id[p]], v_gd[p], sem_g[p]).wait()
            pltpu.make_async_copy(h_hbm.at[v_is[p]], v_h[p], sem_g[p]).wait()

        def sstart(p):
            pltpu.async_copy(v_cb[p], s_tbl.at[v_id[p]], sem_s[p], add=True)

        def swait(p):
            pltpu.make_async_copy(v_cb[p], s_tbl.at[v_id[p]], sem_s[p]).wait()

        def zero_rows(ref, rows):
            def zb(r, _):
                for o in range(tblw // 16):
                    ref[r, pl.ds(o * 16, 16)] = z16
                return 0
            lax.fori_loop(0, rows, zb, 0)

        zero_rows(v_z, 128)
        for p in range(2):
            zero_rows(v_cb[p], C)
            def zid(r, _):
                v_id[p][pl.ds(r * 16, 16)] = zi16
                return 0
            lax.fori_loop(0, C // 16, zid, 0)
        for j in range(RPS // 128):
            pltpu.sync_copy(v_z, s_tbl.at[pl.ds(sid * RPS + j * 128, 128)])
        plsc.subcore_barrier()
        sstart(1)
        istart(0, 0)
        iwait(0, 0)
        gstart(0)

        def compute(p):
            v_gs_p, v_gd_p, v_h_p, v_cb_p = v_gs[p], v_gd[p], v_h[p], v_cb[p]

            def wp(t, _):
                pv = t * 16 + iot
                if heads == 8:
                    row = pv >> 3
                    hd = pv & 7
                else:
                    row = pv
                    hd = iot & 0
                av = plsc.load_gather(v_gs_p, [row, hd])
                bv = plsc.load_gather(v_gd_p, [row, hd + 8])
                e = av + bv
                e = jnp.where(e > 0, e, 0.2 * e)
                plsc.store_scatter(v_cb_p, [row, hd], jnp.exp(e))
                return 0
            lax.fori_loop(0, C * heads // 16, wp, 0, unroll=4)

            def mp(c, _):
                cs = jnp.zeros((16,), jnp.int32) + c
                for k2 in range(kv):
                    wv = plsc.load_gather(v_cb_p, [cs, tmpls[k2]])
                    hv = v_h_p[c, pl.ds(k2 * 16, 16)]
                    v_cb_p[c, pl.ds(8 + k2 * 16, 16)] = wv * hv
                return 0
            lax.fori_loop(0, C, mp, 0, unroll=4)

        def step(g, p, q):
            swait(q)
            gnxt = jnp.minimum(g + 1, G - 1)
            istart(gnxt, q)
            gwait(p)
            iwait(gnxt, q)
            gstart(q)
            compute(p)
            sstart(p)

        def pair(i, _):
            step(2 * i, 0, 1)
            step(2 * i + 1, 1, 0)
            return 0
        lax.fori_loop(0, G // 2, pair, 0)
        swait(1)
        gwait(0)
        plsc.subcore_barrier()
        for j in range(RPS // 128):
            r0 = pl.multiple_of(sid * RPS + j * 128, 128)
            pltpu.sync_copy(s_tbl.at[pl.ds(r0, 128)], v_z)
            pltpu.sync_copy(v_z, out_hbm.at[cid, pl.ds(r0, 128)])

    return k(srcv, dstv, att, feat)


def _edge_pass1bf(srcv, dstv, att, feat_eo, zden, znum):
    mesh = plsc.VectorSubcoreMesh(**_SC_MESH)

    @functools.partial(
        pl.kernel,
        out_type=[jax.ShapeDtypeStruct((2, NPAD, 8), jnp.float32),
                  jax.ShapeDtypeStruct((2, NPAD, 64), jnp.bfloat16)],
        mesh=mesh,
        compiler_params=_SC_PARAMS,
        scratch_types=[
            [pltpu.VMEM((C,), jnp.int32)] * 2,
            [pltpu.VMEM((C,), jnp.int32)] * 2,
            [pltpu.VMEM((C, 16), jnp.float32)] * 2,
            [pltpu.VMEM((C, 16), jnp.float32)] * 2,
            [pltpu.VMEM((C, 64), jnp.float32)] * 2,
            [pltpu.VMEM((C, 8), jnp.float32)] * 2,
            [pltpu.VMEM((C, 64), jnp.bfloat16)] * 2,
            [pltpu.SemaphoreType.DMA] * 2,
            [pltpu.SemaphoreType.DMA] * 2,
            [pltpu.SemaphoreType.DMA] * 2,
            pltpu.VMEM_SHARED((NPAD, 8), jnp.float32),
            pltpu.VMEM_SHARED((NPAD, 64), jnp.bfloat16),
        ],
    )
    def k(src_hbm, dst_hbm, a_hbm, h_hbm, zd_hbm, zn_hbm, oden_hbm, onum_hbm,
          v_is, v_id, v_gs, v_gd, v_h, v_db, v_nb,
          sem_i, sem_g, sem_s, s_dt, s_nt):
        cid = lax.axis_index("c")
        sid = lax.axis_index("s")
        tile = cid * 16 + sid
        z16 = jnp.zeros((16,), jnp.float32)
        zi16 = jnp.zeros((16,), jnp.int32)
        zb32 = jnp.zeros((32,), jnp.bfloat16)
        iot = lax.iota(jnp.int32, 16)

        def ebase(g):
            return tile * EPT + g * C

        def istart(g, p):
            pltpu.async_copy(src_hbm.at[pl.ds(ebase(g), C)], v_is[p], sem_i[p])
            pltpu.async_copy(dst_hbm.at[pl.ds(ebase(g), C)], v_id[p], sem_i[p])

        def iwait(g, p):
            pltpu.make_async_copy(
                src_hbm.at[pl.ds(ebase(g), C)], v_is[p], sem_i[p]).wait()
            pltpu.make_async_copy(
                dst_hbm.at[pl.ds(ebase(g), C)], v_id[p], sem_i[p]).wait()

        def gstart(p):
            pltpu.async_copy(a_hbm.at[v_is[p]], v_gs[p], sem_g[p])
            pltpu.async_copy(a_hbm.at[v_id[p]], v_gd[p], sem_g[p])
            pltpu.async_copy(h_hbm.at[v_is[p]], v_h[p], sem_g[p])

        def gwait(p):
            pltpu.make_async_copy(a_hbm.at[v_is[p]], v_gs[p], sem_g[p]).wait()
            pltpu.make_async_copy(a_hbm.at[v_id[p]], v_gd[p], sem_g[p]).wait()
            pltpu.make_async_copy(h_hbm.at[v_is[p]], v_h[p], sem_g[p]).wait()

        def sstart(p):
            pltpu.async_copy(v_db[p], s_dt.at[v_id[p]], sem_s[p], add=True)
            pltpu.async_copy(v_nb[p], s_nt.at[v_id[p]], sem_s[p], add=True)

        def swait(p):
            pltpu.make_async_copy(v_db[p], s_dt.at[v_id[p]], sem_s[p]).wait()
            pltpu.make_async_copy(v_nb[p], s_nt.at[v_id[p]], sem_s[p]).wait()

        for j in range(RPS // 128):
            r0 = sid * RPS + j * 128
            pltpu.sync_copy(zd_hbm, s_dt.at[pl.ds(r0, 128)])
            pltpu.sync_copy(zn_hbm, s_nt.at[pl.ds(r0, 128)])
        def zb1(t, _):
            pv = t * 16 + iot
            plsc.store_scatter(v_db[1], [pv >> 3, pv & 7], z16)
            return 0
        lax.fori_loop(0, C * 8 // 16, zb1, 0)
        def zb2(r, _):
            v_nb[1][r, pl.ds(0, 32)] = zb32
            v_nb[1][r, pl.ds(32, 32)] = zb32
            return 0
        lax.fori_loop(0, C, zb2, 0)
        def zb3(r, _):
            v_id[1][pl.ds(r * 16, 16)] = zi16
            return 0
        lax.fori_loop(0, C // 16, zb3, 0)
        plsc.subcore_barrier()
        sstart(1)
        istart(0, 0)
        iwait(0, 0)
        gstart(0)

        def compute(p):
            v_gs_p, v_gd_p, v_h_p = v_gs[p], v_gd[p], v_h[p]
            v_db_p, v_nb_p = v_db[p], v_nb[p]

            def wp(t, _):
                pv = t * 16 + iot
                row = pv >> 3
                hd = pv & 7
                av = plsc.load_gather(v_gs_p, [row, hd])
                bv = plsc.load_gather(v_gd_p, [row, hd + 8])
                e = av + bv
                e = jnp.where(e > 0, e, 0.2 * e)
                plsc.store_scatter(v_db_p, [row, hd], jnp.exp(e))
                return 0
            lax.fori_loop(0, C * 8 // 16, wp, 0, unroll=4)

            def mp(c, _):
                cs = jnp.zeros((16,), jnp.int32) + c
                for k2 in range(2):
                    tmpl = (iot >> 2) + 4 * k2
                    wv = plsc.load_gather(v_db_p, [cs, tmpl])
                    av = v_h_p[c, pl.ds(16 * k2, 16)]
                    bv = v_h_p[c, pl.ds(32 + 16 * k2, 16)]
                    ia = plsc.bitcast(wv * av, jnp.int32)
                    ib = plsc.bitcast(wv * bv, jnp.int32)
                    ra = lax.shift_right_logical(ia + 0x8000, 16)
                    rb = (ib + 0x8000) & jnp.int32(-65536)
                    v_nb_p[c, pl.ds(32 * k2, 32)] = plsc.bitcast(
                        ra | rb, jnp.bfloat16)
                return 0
            lax.fori_loop(0, C, mp, 0, unroll=4)

        def step(g, p, q):
            swait(q)
            gnxt = jnp.minimum(g + 1, G - 1)
            istart(gnxt, q)
            gwait(p)
            iwait(gnxt, q)
            gstart(q)
            compute(p)
            sstart(p)

        def pair(i, _):
            step(2 * i, 0, 1)
            step(2 * i + 1, 1, 0)
            return 0
        lax.fori_loop(0, G // 2, pair, 0)
        swait(1)
        gwait(0)
        plsc.subcore_barrier()
        for j in range(RPS // 128):
            r0 = pl.multiple_of(sid * RPS + j * 128, 128)
            pltpu.sync_copy(s_dt.at[pl.ds(r0, 128)], v_db[0])
            pltpu.sync_copy(v_db[0], oden_hbm.at[cid, pl.ds(r0, 128)])
            pltpu.sync_copy(s_nt.at[pl.ds(r0, 128)], v_nb[0])
            pltpu.sync_copy(v_nb[0], onum_hbm.at[cid, pl.ds(r0, 128)])

    return k(srcv, dstv, att, feat_eo, zden, znum)


def _k1_body(x_ref, w_ref, s_ref, h_ref, a_ref):
    h = jnp.dot(x_ref[...], w_ref[...], preferred_element_type=jnp.float32)
    h_ref[...] = h
    a_ref[...] = jnp.dot(h, s_ref[...], preferred_element_type=jnp.float32)


def _k2_body(d_ref, n_ref, er_ref, b_ref, w2_ref, s2_ref, h2_ref, a2_ref):
    num = n_ref[0].astype(jnp.float32) + n_ref[1].astype(jnp.float32)
    den = jnp.dot(d_ref[0] + d_ref[1], er_ref[...],
                  preferred_element_type=jnp.float32)
    out1 = num / (den + 1e-16) + b_ref[...]
    out1 = jnp.where(out1 > 0, out1, 0.2 * out1)
    h2 = jnp.dot(out1, w2_ref[...], preferred_element_type=jnp.float32)
    h2_ref[...] = h2
    a2_ref[...] = jnp.dot(h2, s2_ref[...], preferred_element_type=jnp.float32)


def _k3_body(p_ref, b_ref, o_ref):
    t = p_ref[0] + p_ref[1]
    z = t[:, 8:24] / (t[:, 0:1] + 1e-16) + b_ref[...]
    m = jnp.max(z, axis=1, keepdims=True)
    o_ref[...] = z - m - jnp.log(jnp.sum(jnp.exp(z - m), axis=1, keepdims=True))


def kernel(x, edge_index, W1, as1, ad1, b1, W2, as2, ad2, b2):
    n = x.shape[0]
    loops = jnp.arange(n, dtype=jnp.int32)
    pad = jnp.full((E_PAD - edge_index.shape[1] - n,), N_NODES, jnp.int32)
    src = jnp.concatenate([edge_index[0].astype(jnp.int32), loops, pad])
    dst = jnp.concatenate([edge_index[1].astype(jnp.int32), loops, pad])
    x_pad = jnp.zeros((NPAD, F_IN), jnp.float32).at[:n].set(x)

    perm = jnp.array([2 * i for i in range(32)] + [2 * i + 1 for i in range(32)])
    W1eo = W1[:, perm]
    s1 = jnp.zeros((64, 16), jnp.float32)
    for hd in range(8):
        s1 = s1.at[hd * 8:(hd + 1) * 8, hd].set(as1[hd])
        s1 = s1.at[hd * 8:(hd + 1) * 8, 8 + hd].set(ad1[hd])
    s1eo = s1[perm, :]
    s2 = jnp.zeros((16, 16), jnp.float32).at[:, 0].set(as2[0]).at[:, 8].set(ad2[0])
    erep = jnp.repeat(jnp.eye(8, dtype=jnp.float32), 8, axis=1)
    zden = jnp.zeros((128, 8), jnp.float32)
    znum = jnp.zeros((128, 64), jnp.bfloat16)

    grid = (NPAD // RB,)
    h1, a1 = pl.pallas_call(
        _k1_body,
        grid=grid,
        in_specs=[
            pl.BlockSpec((RB, F_IN), lambda i: (i, 0)),
            pl.BlockSpec((F_IN, 64), lambda i: (0, 0)),
            pl.BlockSpec((64, 16), lambda i: (0, 0)),
        ],
        out_specs=[
            pl.BlockSpec((RB, 64), lambda i: (i, 0)),
            pl.BlockSpec((RB, 16), lambda i: (i, 0)),
        ],
        out_shape=[
            jax.ShapeDtypeStruct((NPAD, 64), jnp.float32),
            jax.ShapeDtypeStruct((NPAD, 16), jnp.float32),
        ],
    )(x_pad, W1eo, s1eo)

    den1, num1 = _edge_pass1bf(src, dst, a1, h1, zden, znum)

    h2, a2 = pl.pallas_call(
        _k2_body,
        grid=grid,
        in_specs=[
            pl.BlockSpec((2, RB, 8), lambda i: (0, i, 0)),
            pl.BlockSpec((2, RB, 64), lambda i: (0, i, 0)),
            pl.BlockSpec((8, 64), lambda i: (0, 0)),
            pl.BlockSpec((1, 64), lambda i: (0, 0)),
            pl.BlockSpec((64, 16), lambda i: (0, 0)),
            pl.BlockSpec((16, 16), lambda i: (0, 0)),
        ],
        out_specs=[
            pl.BlockSpec((RB, 16), lambda i: (i, 0)),
            pl.BlockSpec((RB, 16), lambda i: (i, 0)),
        ],
        out_shape=[
            jax.ShapeDtypeStruct((NPAD, 16), jnp.float32),
            jax.ShapeDtypeStruct((NPAD, 16), jnp.float32),
        ],
    )(den1, num1, erep, b1.reshape(1, 64), W2, s2)

    p2 = _edge_pass(src, dst, a2, h2, heads=1, cols=16, tblw=32)

    out = pl.pallas_call(
        _k3_body,
        grid=grid,
        in_specs=[
            pl.BlockSpec((2, RB, 32), lambda i: (0, i, 0)),
            pl.BlockSpec((1, 16), lambda i: (0, 0)),
        ],
        out_specs=pl.BlockSpec((RB, 16), lambda i: (i, 0)),
        out_shape=jax.ShapeDtypeStruct((NPAD, 16), jnp.float32),
    )(p2, b2.reshape(1, 16))
    return out[:n]
